# baseline (speedup 1.0000x reference)
import jax
import jax.numpy as jnp
from jax.experimental import pallas as pl
from jax.experimental.pallas import tpu as pltpu

_N_CHUNKS = 8


def _dma_copy_kernel(w_ref, o_ref, sem):
    rows = w_ref.shape[0]
    chunk = rows // _N_CHUNKS
    copies = [
        pltpu.make_async_copy(
            w_ref.at[pl.ds(i * chunk, chunk)],
            o_ref.at[pl.ds(i * chunk, chunk)],
            sem,
        )
        for i in range(_N_CHUNKS)
    ]
    for c in copies:
        c.start()
    for c in copies:
        c.wait()


def kernel(x, W):
    seq_len = x.shape[1]
    rows, dim = W.shape
    out = pl.pallas_call(
        _dma_copy_kernel,
        in_specs=[pl.BlockSpec(memory_space=pl.ANY)],
        out_specs=pl.BlockSpec(memory_space=pl.ANY),
        out_shape=jax.ShapeDtypeStruct((seq_len, dim), W.dtype),
        scratch_shapes=[pltpu.SemaphoreType.DMA],
    )(W)
    return out[None, :, :]


# speedup vs baseline: 41.0677x; 41.0677x over previous
import jax
import jax.numpy as jnp
from jax.experimental import pallas as pl

_BLOCK_ROWS = 512


def _copy_kernel(w_ref, o_ref):
    o_ref[...] = w_ref[...]


def kernel(x, W):
    seq_len = x.shape[1]
    rows, dim = W.shape
    grid = (seq_len // _BLOCK_ROWS,)
    out = pl.pallas_call(
        _copy_kernel,
        grid=grid,
        in_specs=[pl.BlockSpec((_BLOCK_ROWS, dim), lambda i: (i, 0))],
        out_specs=pl.BlockSpec((_BLOCK_ROWS, dim), lambda i: (i, 0)),
        out_shape=jax.ShapeDtypeStruct((seq_len, dim), W.dtype),
    )(W)
    return out[None, :, :]


# speedup vs baseline: 48.4749x; 1.1804x over previous
import jax
import jax.numpy as jnp
from jax.experimental import pallas as pl

_BLOCK_ROWS = 2048


def _copy_kernel(w_ref, o_ref):
    o_ref[...] = w_ref[...]


def kernel(x, W):
    seq_len = x.shape[1]
    rows, dim = W.shape
    grid = (seq_len // _BLOCK_ROWS,)
    out = pl.pallas_call(
        _copy_kernel,
        grid=grid,
        in_specs=[pl.BlockSpec((_BLOCK_ROWS, dim), lambda i: (i, 0))],
        out_specs=pl.BlockSpec((_BLOCK_ROWS, dim), lambda i: (i, 0)),
        out_shape=jax.ShapeDtypeStruct((seq_len, dim), W.dtype),
    )(W)
    return out[None, :, :]
